# baseline (speedup 1.0000x reference)
import functools

import jax
import jax.numpy as jnp
from jax import lax
from jax.experimental import pallas as pl
from jax.experimental.pallas import tpu as pltpu
from jax.experimental.pallas import tpu_sc as plsc

B, S, L, D = 1024, 50, 20, 64
V = 100000
TB = 512
BS = B * S

NC, NS = 2, 16
NW = NC * NS
HALF = BS // 2
COLS_W = HALF // NW
CCH = 80
NCHUNK = COLS_W // CCH
LG = L + 1

R_TC = 3200
G_TC = HALF // R_TC


def _sc_gather_sum(idx_t, token_table, comb_table):
  mesh = plsc.VectorSubcoreMesh(core_axis_name="c", subcore_axis_name="s")

  @functools.partial(
      pl.kernel,
      mesh=mesh,
      out_type=jax.ShapeDtypeStruct((HALF, D), jnp.float32),
      scratch_types=[
          pltpu.VMEM((LG, COLS_W), jnp.int32),
          pltpu.VMEM((2, CCH, D), jnp.float32),
          pltpu.SemaphoreType.DMA,
          pltpu.SemaphoreType.DMA,
          pltpu.SemaphoreType.DMA,
          pltpu.SemaphoreType.DMA,
          pltpu.SemaphoreType.DMA,
          pltpu.SemaphoreType.DMA,
          pltpu.SemaphoreType.DMA,
      ],
      compiler_params=pltpu.CompilerParams(use_tc_tiling_on_sc=False),
  )
  def body(idx_hbm, ttab_hbm, ctab_hbm, out_hbm, idx_v, acc_v, sl0_0, sl0_1,
           sadd_0, sadd_1, sout_0, sout_1, s_stage):
    s_l0 = (sl0_0, sl0_1)
    s_add = (sadd_0, sadd_1)
    s_out = (sout_0, sout_1)
    wid = lax.axis_index("s") * NC + lax.axis_index("c")
    base = wid * COLS_W

    def l0_copy(off, p):
      return pltpu.make_async_copy(
          ttab_hbm.at[idx_v.at[0, pl.ds(off, CCH)]], acc_v.at[p], s_l0[p])

    def out_copy(off, p):
      return pltpu.make_async_copy(
          acc_v.at[p], out_hbm.at[pl.ds(base + off, CCH)], s_out[p])

    pltpu.async_copy(idx_hbm.at[0, pl.ds(base, COLS_W)], idx_v.at[0],
                     s_stage).wait()
    l0_copy(0, 0).start()
    stage = [
        pltpu.async_copy(idx_hbm.at[l, pl.ds(base, COLS_W)], idx_v.at[l],
                         s_stage) for l in range(1, LG)
    ]
    for dsc in stage:
      dsc.wait()

    def step(k2, carry):
      for p in range(2):
        q = 1 - p
        k = k2 * 2 + p
        off = k * CCH
        l0_copy(off, p).wait()
        descs = []
        for l in range(1, L):
          descs.append(
              pltpu.async_copy(
                  ttab_hbm.at[idx_v.at[l, pl.ds(off, CCH)]], acc_v.at[p],
                  s_add[p], add=True))
        descs.append(
            pltpu.async_copy(
                ctab_hbm.at[idx_v.at[L, pl.ds(off, CCH)]], acc_v.at[p],
                s_add[p], add=True))
        @pl.when(k >= 1)
        def _():
          out_copy((k - 1) * CCH, q).wait()

        @pl.when(k + 1 < NCHUNK)
        def _():
          l0_copy((k + 1) * CCH, q).start()

        for dsc in descs:
          dsc.wait()
        out_copy(off, p).start()
      return carry

    lax.fori_loop(0, NCHUNK // 2, step, 0)
    out_copy((NCHUNK - 1) * CCH, 1).wait()

  return body(idx_t, token_table, comb_table)


def _tc_finish(pre, pos_tiled, gamma2, beta2, W, b2, phase, y_prev=None):
  bb = R_TC // S

  def body(*refs):
    if phase == 1:
      refs = refs[1:]
    pre_ref, pos_ref, g_ref, be_ref, w_ref, b_ref, o_ref = refs
    x = pre_ref[...] * (1.0 / L) + pos_ref[...]
    mu = jnp.mean(x, axis=1, keepdims=True)
    xc = x - mu
    var = jnp.mean(xc * xc, axis=1, keepdims=True)
    nx = xc * lax.rsqrt(var + 1e-5) * g_ref[...] + be_ref[...]
    y = lax.dot_general(nx, w_ref[...], (((1,), (1,)), ((), ())),
                        preferred_element_type=jnp.float32,
                        precision=lax.Precision.HIGHEST)
    o_ref[...] = (y + b_ref[...]).reshape(bb, S, D)

  blk0 = phase * (B // 2 // bb)
  specs = [
      pl.BlockSpec((R_TC, D), lambda i: (i, 0)),
      pl.BlockSpec((R_TC, D), lambda i: (0, 0)),
      pl.BlockSpec((1, D), lambda i: (0, 0)),
      pl.BlockSpec((1, D), lambda i: (0, 0)),
      pl.BlockSpec((D, D), lambda i: (0, 0)),
      pl.BlockSpec((1, D), lambda i: (0, 0)),
  ]
  args = [pre, pos_tiled, gamma2, beta2, W, b2]
  aliases = {}
  if phase == 1:
    specs = [pl.BlockSpec(memory_space=pl.ANY)] + specs
    args = [y_prev] + args
    aliases = {0: 0}
  return pl.pallas_call(
      body,
      grid=(G_TC,),
      in_specs=specs,
      out_specs=pl.BlockSpec((bb, S, D), lambda i: (i + blk0, 0, 0)),
      out_shape=jax.ShapeDtypeStruct((B, S, D), jnp.float32),
      input_output_aliases=aliases,
  )(*args)


def kernel(token_ids, section, temporality, negated, timestamp_bucket,
           token_table, section_table, temporality_table, negation_table,
           position_table, timestamp_table, ln_gamma, ln_beta, W, b):
  comb_table = (section_table[:, None, None, None, :]
                + temporality_table[None, :, None, None, :]
                + negation_table[None, None, :, None, :]
                + timestamp_table[None, None, None, :, :]
                ).reshape(6 * 3 * 2 * TB, D) * float(L)

  comb_idx = (((section.astype(jnp.int32) * 3 + temporality.astype(jnp.int32))
               * 2 + negated.astype(jnp.int32)) * TB
              + timestamp_bucket.astype(jnp.int32)).reshape(BS)

  idx_t = jnp.concatenate([
      token_ids.astype(jnp.int32).reshape(BS, L), comb_idx[:, None]
  ], axis=1).T

  pos_tiled = jnp.tile(position_table, (R_TC // S, 1))
  g2 = ln_gamma.reshape(1, D)
  be2 = ln_beta.reshape(1, D)
  b2 = b.reshape(1, D)

  pre1 = _sc_gather_sum(idx_t[:, :HALF], token_table, comb_table)
  pre2 = _sc_gather_sum(idx_t[:, HALF:], token_table, comb_table)
  y1 = _tc_finish(pre1, pos_tiled, g2, be2, W, b2, phase=0)
  tokens = _tc_finish(pre2, pos_tiled, g2, be2, W, b2, phase=1, y_prev=y1)
  padding_mask = jnp.zeros((B, S), dtype=bool)
  return tokens, padding_mask


# speedup vs baseline: 1.0235x; 1.0235x over previous
import functools

import jax
import jax.numpy as jnp
from jax import lax
from jax.experimental import pallas as pl
from jax.experimental.pallas import tpu as pltpu
from jax.experimental.pallas import tpu_sc as plsc

B, S, L, D = 1024, 50, 20, 64
V = 100000
TB = 512
BS = B * S

NC, NS = 2, 16
NW = NC * NS
HALF = BS // 2
COLS_W = HALF // NW
CCH = 80
NCHUNK = COLS_W // CCH
LG = L + 1

R_TC = 3200
G_TC = HALF // R_TC


def _sc_gather_sum(idx_t, token_table, comb_table):
  mesh = plsc.VectorSubcoreMesh(core_axis_name="c", subcore_axis_name="s")

  @functools.partial(
      pl.kernel,
      mesh=mesh,
      out_type=jax.ShapeDtypeStruct((HALF, D), jnp.float32),
      scratch_types=[
          pltpu.VMEM((LG, COLS_W), jnp.int32),
          pltpu.VMEM((2, CCH, D), jnp.float32),
          pltpu.SemaphoreType.DMA,
          pltpu.SemaphoreType.DMA,
          pltpu.SemaphoreType.DMA,
          pltpu.SemaphoreType.DMA,
          pltpu.SemaphoreType.DMA,
          pltpu.SemaphoreType.DMA,
          pltpu.SemaphoreType.DMA,
      ],
      compiler_params=pltpu.CompilerParams(use_tc_tiling_on_sc=False),
  )
  def body(idx_hbm, ttab_hbm, ctab_hbm, out_hbm, idx_v, acc_v, sl0_0, sl0_1,
           sadd_0, sadd_1, sout_0, sout_1, s_stage):
    s_l0 = (sl0_0, sl0_1)
    s_add = (sadd_0, sadd_1)
    s_out = (sout_0, sout_1)
    wid = lax.axis_index("s") * NC + lax.axis_index("c")
    base = wid * COLS_W

    def l0_copy(off, p):
      return pltpu.make_async_copy(
          ttab_hbm.at[idx_v.at[0, pl.ds(off, CCH)]], acc_v.at[p], s_l0[p])

    def out_copy(off, p):
      return pltpu.make_async_copy(
          acc_v.at[p], out_hbm.at[pl.ds(base + off, CCH)], s_out[p])

    pltpu.async_copy(idx_hbm.at[0, pl.ds(base, COLS_W)], idx_v.at[0],
                     s_stage).wait()
    l0_copy(0, 0).start()
    stage = [
        pltpu.async_copy(idx_hbm.at[l, pl.ds(base, COLS_W)], idx_v.at[l],
                         s_stage) for l in range(1, LG)
    ]
    for dsc in stage:
      dsc.wait()

    def step(k2, carry):
      for p in range(2):
        q = 1 - p
        k = k2 * 2 + p
        off = k * CCH
        l0_copy(off, p).wait()
        descs = []
        for l in range(1, L):
          descs.append(
              pltpu.async_copy(
                  ttab_hbm.at[idx_v.at[l, pl.ds(off, CCH)]], acc_v.at[p],
                  s_add[p], add=True))
        descs.append(
            pltpu.async_copy(
                ctab_hbm.at[idx_v.at[L, pl.ds(off, CCH)]], acc_v.at[p],
                s_add[p], add=True))
        @pl.when(k >= 1)
        def _():
          out_copy((k - 1) * CCH, q).wait()

        @pl.when(k + 1 < NCHUNK)
        def _():
          l0_copy((k + 1) * CCH, q).start()

        for dsc in descs:
          dsc.wait()
        out_copy(off, p).start()
      return carry

    lax.fori_loop(0, NCHUNK // 2, step, 0)
    out_copy((NCHUNK - 1) * CCH, 1).wait()

  return body(idx_t, token_table, comb_table)


def _tc_finish(pre, pos_tiled, gamma2, beta2, W, b2):
  def body(pre_ref, pos_ref, g_ref, be_ref, w_ref, b_ref, o_ref):
    x = pre_ref[...] * (1.0 / L) + pos_ref[...]
    mu = jnp.mean(x, axis=1, keepdims=True)
    xc = x - mu
    var = jnp.mean(xc * xc, axis=1, keepdims=True)
    nx = xc * lax.rsqrt(var + 1e-5) * g_ref[...] + be_ref[...]
    y = lax.dot_general(nx, w_ref[...], (((1,), (1,)), ((), ())),
                        preferred_element_type=jnp.float32,
                        precision=lax.Precision.HIGHEST)
    o_ref[...] = (y + b_ref[...]).reshape(R_TC // S, S, D)

  return pl.pallas_call(
      body,
      grid=(G_TC,),
      in_specs=[
          pl.BlockSpec((R_TC, D), lambda i: (i, 0)),
          pl.BlockSpec((R_TC, D), lambda i: (0, 0)),
          pl.BlockSpec((1, D), lambda i: (0, 0)),
          pl.BlockSpec((1, D), lambda i: (0, 0)),
          pl.BlockSpec((D, D), lambda i: (0, 0)),
          pl.BlockSpec((1, D), lambda i: (0, 0)),
      ],
      out_specs=pl.BlockSpec((R_TC // S, S, D), lambda i: (i, 0, 0)),
      out_shape=jax.ShapeDtypeStruct((B // 2, S, D), jnp.float32),
  )(pre, pos_tiled, gamma2, beta2, W, b2)


def kernel(token_ids, section, temporality, negated, timestamp_bucket,
           token_table, section_table, temporality_table, negation_table,
           position_table, timestamp_table, ln_gamma, ln_beta, W, b):
  comb_table = (section_table[:, None, None, None, :]
                + temporality_table[None, :, None, None, :]
                + negation_table[None, None, :, None, :]
                + timestamp_table[None, None, None, :, :]
                ).reshape(6 * 3 * 2 * TB, D) * float(L)

  comb_idx = (((section.astype(jnp.int32) * 3 + temporality.astype(jnp.int32))
               * 2 + negated.astype(jnp.int32)) * TB
              + timestamp_bucket.astype(jnp.int32)).reshape(BS)

  idx_t = jnp.concatenate([
      token_ids.astype(jnp.int32).reshape(BS, L), comb_idx[:, None]
  ], axis=1).T

  pos_tiled = jnp.tile(position_table, (R_TC // S, 1))
  g2 = ln_gamma.reshape(1, D)
  be2 = ln_beta.reshape(1, D)
  b2 = b.reshape(1, D)

  pre1 = _sc_gather_sum(idx_t[:, :HALF], token_table, comb_table)
  pre2 = _sc_gather_sum(idx_t[:, HALF:], token_table, comb_table)
  y1 = _tc_finish(pre1, pos_tiled, g2, be2, W, b2)
  y2 = _tc_finish(pre2, pos_tiled, g2, be2, W, b2)
  tokens = jnp.concatenate([y1, y2], axis=0)
  padding_mask = jnp.zeros((B, S), dtype=bool)
  return tokens, padding_mask
